# baseline (speedup 1.0000x reference)
import functools

import jax
import jax.numpy as jnp
import numpy as np
from jax import lax
from jax.experimental import pallas as pl
from jax.experimental.pallas import tpu as pltpu
from jax.experimental.pallas import tpu_sc as plsc

VOCAB = 100000
D_MODEL = 128
BATCH = 4096
SEQ = 200
SCALE = float(np.sqrt(D_MODEL))

HALF = 2
GLEN = SEQ // HALF


def _positional_encoding(length, depth):
    positions = np.arange(length)[:, np.newaxis]
    depths = np.arange(depth // 2)[np.newaxis, :] / (depth // 2)
    angle_rates = 1.0 / (10000 ** depths)
    angle_rads = positions * angle_rates
    return np.concatenate(
        [np.sin(angle_rads), np.cos(angle_rads)], axis=-1
    ).astype(np.float32)


_PE = _positional_encoding(SEQ, D_MODEL)


def _make_sc_kernel():
    info = plsc.get_sparse_core_info()
    nc, ns, lanes = info.num_cores, info.num_subcores, info.num_lanes
    nw = nc * ns
    seq_per_w = BATCH // nw
    iters = seq_per_w // 2
    nvec = D_MODEL // lanes
    mesh = plsc.VectorSubcoreMesh(core_axis_name="c", subcore_axis_name="s")

    @functools.partial(
        pl.kernel,
        mesh=mesh,
        out_type=jax.ShapeDtypeStruct((BATCH, SEQ, D_MODEL), jnp.float32),
        scratch_types=[
            pltpu.VMEM((SEQ, D_MODEL), jnp.float32),
            pltpu.VMEM((SEQ, D_MODEL), jnp.float32),
            pltpu.VMEM((SEQ, D_MODEL), jnp.float32),
            pltpu.VMEM((HALF, GLEN), jnp.int32),
            pltpu.VMEM((HALF, GLEN), jnp.int32),
        ]
        + [pltpu.SemaphoreType.DMA] * 6,
    )
    def k(idx_hbm, table_hbm, pe_hbm, out_hbm,
          pe_v, r0, r1, i0, i1, g0, g1, s0, s1, m0, m1):
        rows = (r0, r1)
        idx = (i0, i1)
        gsem = (g0, g1)
        ssem = (s0, s1)
        isem = (m0, m1)
        wid = lax.axis_index("s") * nc + lax.axis_index("c")
        sbase = wid * seq_per_w
        pltpu.sync_copy(pe_hbm, pe_v)
        pltpu.sync_copy(idx_hbm.at[sbase], idx[0])
        pltpu.sync_copy(idx_hbm.at[sbase + 1], idx[1])
        for h in range(HALF):
            pltpu.async_copy(
                table_hbm.at[idx[0].at[h]],
                rows[0].at[pl.ds(h * GLEN, GLEN)],
                gsem[0],
            )

        def fire_gathers(j, seq):
            for h in range(HALF):
                pltpu.async_copy(
                    table_hbm.at[idx[j].at[h]],
                    rows[j].at[pl.ds(h * GLEN, GLEN)],
                    gsem[j],
                )

        def drain_gathers(j):
            pltpu.make_async_copy(out_hbm.at[sbase], rows[j], gsem[j]).wait()

        def drain_store(j):
            pltpu.make_async_copy(rows[j], out_hbm.at[sbase], ssem[j]).wait()

        def drain_idx(j):
            pltpu.make_async_copy(idx_hbm.at[sbase], idx[j], isem[j]).wait()

        def compute_and_store(j, seq):
            def rbody(t, c2):
                for r in range(4):
                    jj = 4 * t + r
                    for v in range(nvec):
                        sl = pl.ds(v * lanes, lanes)
                        rows[j][jj, sl] = (
                            rows[j][jj, sl] * SCALE + pe_v[jj, sl]
                        )
                return c2

            lax.fori_loop(0, SEQ // 4, rbody, 0)
            pltpu.async_copy(rows[j], out_hbm.at[seq], ssem[j])

        def body(i, carry):
            @pl.when(i > 0)
            def _drains0():
                drain_store(1)
                drain_idx(1)

            fire_gathers(1, sbase + 2 * i + 1)
            drain_gathers(0)

            @pl.when(i < iters - 1)
            def _idx0():
                pltpu.async_copy(
                    idx_hbm.at[sbase + 2 * i + 2], idx[0], isem[0]
                )

            compute_and_store(0, sbase + 2 * i)

            drain_store(0)

            @pl.when(i < iters - 1)
            def _g1():
                drain_idx(0)
                fire_gathers(0, sbase + 2 * i + 2)

            drain_gathers(1)

            @pl.when(i < iters - 1)
            def _idx1():
                pltpu.async_copy(
                    idx_hbm.at[sbase + 2 * i + 3], idx[1], isem[1]
                )

            compute_and_store(1, sbase + 2 * i + 1)
            return carry

        lax.fori_loop(0, iters, body, 0)
        drain_store(1)

    return k


_sc_kernel = _make_sc_kernel()


def kernel(inputs, table):
    idx = inputs.reshape(BATCH, HALF, GLEN)
    return _sc_kernel(idx, table, jnp.asarray(_PE))
